# baseline (speedup 1.0000x reference)
import math

import jax
import jax.numpy as jnp
from jax import lax
from jax.experimental import pallas as pl
from jax.experimental.pallas import tpu as pltpu
from jax.experimental.pallas import tpu_sc as plsc

_D = 64
_LANES = 16
_NC = 2
_NS = 16
_NW = _NC * _NS
_K = 4
_SCALE = math.sqrt(_D)


def _body(x_ref, table_ref, out_ref, idx_v, idx_t, gbuf, tbuf, gsems, osems):
    wid = lax.axis_index("s") * _NC + lax.axis_index("c")
    bpw = x_ref.shape[0] // _NW
    seq = x_ref.shape[1]
    b0 = wid * bpw

    pltpu.sync_copy(x_ref.at[pl.ds(b0, bpw)], idx_v)

    iota = lax.iota(jnp.int32, _LANES)
    zeros = jnp.zeros((_LANES,), jnp.int32)

    @pl.loop(0, seq)
    def _tr(s):
        svec = zeros + s
        for c in range(bpw // _LANES):
            vals = plsc.load_gather(idx_v, [iota + c * _LANES, svec])
            idx_t[s, pl.ds(c * _LANES, _LANES)] = vals

    def start_gather(k, s):
        pltpu.async_copy(table_ref.at[idx_t.at[s]], gbuf.at[k], gsems.at[k])

    def wait_gather(k, s):
        pltpu.make_async_copy(
            table_ref.at[idx_t.at[s]], gbuf.at[k], gsems.at[k]
        ).wait()

    def start_out(k, s):
        pltpu.async_copy(
            tbuf.at[k], out_ref.at[s, :, pl.ds(b0, bpw)], osems.at[k]
        )

    def wait_out(k):
        pltpu.make_async_copy(
            tbuf.at[k], out_ref.at[0, :, pl.ds(0, bpw)], osems.at[k]
        ).wait()

    for k in range(_K):
        start_gather(k, k)

    @pl.loop(0, seq // _K)
    def _rounds(r):
        for k in range(_K):
            s = r * _K + k
            wait_gather(k, s)

            @pl.when(s >= _K)
            def _():
                wait_out(k)

            @pl.loop(0, _D, unroll=2)
            def _tp(d):
                dvec = zeros + d
                for c in range(bpw // _LANES):
                    vals = plsc.load_gather(
                        gbuf.at[k], [iota + c * _LANES, dvec]
                    )
                    tbuf[k, d, pl.ds(c * _LANES, _LANES)] = vals * _SCALE

            @pl.when(s + _K < seq)
            def _():
                start_gather(k, s + _K)

            start_out(k, s)

    for k in range(_K):
        wait_out(k)


@jax.jit
def kernel(x, table):
    batch, seq = x.shape
    bpw = batch // _NW

    fn = pl.kernel(
        _body,
        out_type=jax.ShapeDtypeStruct((seq, _D, batch), jnp.float32),
        mesh=plsc.VectorSubcoreMesh(core_axis_name="c", subcore_axis_name="s"),
        scratch_types=[
            pltpu.VMEM((bpw, seq), jnp.int32),
            pltpu.VMEM((seq, bpw), jnp.int32),
            pltpu.VMEM((_K, bpw, _D), jnp.float32),
            pltpu.VMEM((_K, _D, bpw), jnp.float32),
            pltpu.SemaphoreType.DMA((_K,)),
            pltpu.SemaphoreType.DMA((_K,)),
        ],
        compiler_params=pltpu.CompilerParams(
            use_tc_tiling_on_sc=False, needs_layout_passes=False
        ),
    )
    out = fn(x.astype(jnp.int32), table)
    return jnp.transpose(out, (2, 0, 1))


# speedup vs baseline: 1.5897x; 1.5897x over previous
import math

import jax
import jax.numpy as jnp
from jax import lax
from jax.experimental import pallas as pl
from jax.experimental.pallas import tpu as pltpu
from jax.experimental.pallas import tpu_sc as plsc

_D = 64
_LANES = 16
_NC = 2
_NS = 16
_NW = _NC * _NS
_K = 4
_SCALE = math.sqrt(_D)


def _body(x_ref, table_ref, out_ref, idx_v, idx_t, gbuf, tbuf, gsems, osems):
    wid = lax.axis_index("s") * _NC + lax.axis_index("c")
    bpw = x_ref.shape[0] // _NW
    seq = x_ref.shape[1]
    b0 = wid * bpw

    pltpu.sync_copy(x_ref.at[pl.ds(b0, bpw)], idx_v)

    iota = lax.iota(jnp.int32, _LANES)
    zeros = jnp.zeros((_LANES,), jnp.int32)
    rots = [jnp.bitwise_and(iota + j, _LANES - 1) for j in range(_LANES)]

    @pl.loop(0, seq)
    def _tr(s):
        svec = zeros + s
        for c in range(bpw // _LANES):
            vals = plsc.load_gather(idx_v, [iota + c * _LANES, svec])
            idx_t[s, pl.ds(c * _LANES, _LANES)] = vals

    def start_gather(k, s):
        pltpu.async_copy(table_ref.at[idx_t.at[s]], gbuf.at[k], gsems.at[k])

    def wait_gather(k, s):
        pltpu.make_async_copy(
            table_ref.at[idx_t.at[s]], gbuf.at[k], gsems.at[k]
        ).wait()

    def start_out(k, s):
        pltpu.async_copy(
            tbuf.at[k], out_ref.at[s, :, pl.ds(b0, bpw)], osems.at[k]
        )

    def wait_out(k):
        pltpu.make_async_copy(
            tbuf.at[k], out_ref.at[0, :, pl.ds(0, bpw)], osems.at[k]
        ).wait()

    for k in range(_K):
        start_gather(k, k)

    @pl.loop(0, seq // _K)
    def _rounds(r):
        for k in range(_K):
            s = r * _K + k
            wait_gather(k, s)

            @pl.when(s >= _K)
            def _():
                wait_out(k)

            @pl.loop(0, _D // _LANES)
            def _tp(dc):
                dbase = dc * _LANES
                dvecs = [rots[j] + dbase for j in range(_LANES)]
                for c in range(bpw // _LANES):
                    ivec = iota + c * _LANES
                    for j in range(_LANES):
                        v = plsc.load_gather(gbuf.at[k], [ivec, dvecs[j]])
                        plsc.store_scatter(
                            tbuf.at[k], [dvecs[j], ivec], v * _SCALE
                        )

            @pl.when(s + _K < seq)
            def _():
                start_gather(k, s + _K)

            start_out(k, s)

    for k in range(_K):
        wait_out(k)


@jax.jit
def kernel(x, table):
    batch, seq = x.shape
    bpw = batch // _NW

    fn = pl.kernel(
        _body,
        out_type=jax.ShapeDtypeStruct((seq, _D, batch), jnp.float32),
        mesh=plsc.VectorSubcoreMesh(core_axis_name="c", subcore_axis_name="s"),
        scratch_types=[
            pltpu.VMEM((bpw, seq), jnp.int32),
            pltpu.VMEM((seq, bpw), jnp.int32),
            pltpu.VMEM((_K, bpw, _D), jnp.float32),
            pltpu.VMEM((_K, _D, bpw), jnp.float32),
            pltpu.SemaphoreType.DMA((_K,)),
            pltpu.SemaphoreType.DMA((_K,)),
        ],
        compiler_params=pltpu.CompilerParams(
            use_tc_tiling_on_sc=False, needs_layout_passes=False
        ),
    )
    out = fn(x.astype(jnp.int32), table)
    return jnp.transpose(out, (2, 0, 1))


# speedup vs baseline: 1.9700x; 1.2392x over previous
import math

import jax
import jax.numpy as jnp
from jax import lax
from jax.experimental import pallas as pl
from jax.experimental.pallas import tpu as pltpu
from jax.experimental.pallas import tpu_sc as plsc

_D = 64
_LANES = 16
_NC = 2
_NS = 16
_NW = _NC * _NS
_K = 4
_SCALE = math.sqrt(_D)


def _body(xt_ref, table_ref, out_ref, idx_v, gbuf, tbuf, gsems, osems):
    wid = lax.axis_index("s") * _NC + lax.axis_index("c")
    seq = xt_ref.shape[0]
    bpw = xt_ref.shape[1] // _NW
    b0 = wid * bpw

    pltpu.sync_copy(xt_ref.at[:, pl.ds(b0, bpw)], idx_v)

    iota = lax.iota(jnp.int32, _LANES)
    rots = [jnp.bitwise_and(iota + j, _LANES - 1) for j in range(_LANES)]

    def start_gather(k, s):
        pltpu.async_copy(table_ref.at[idx_v.at[s]], gbuf.at[k], gsems.at[k])

    def wait_gather(k, s):
        pltpu.make_async_copy(
            table_ref.at[idx_v.at[s]], gbuf.at[k], gsems.at[k]
        ).wait()

    def start_out(k, s):
        for dt in range(_D // 8):
            pltpu.async_copy(
                tbuf.at[k, pl.ds(dt * 8, 8)],
                out_ref.at[s, dt, wid],
                osems.at[k],
            )

    def wait_out(k):
        for dt in range(_D // 8):
            pltpu.make_async_copy(
                tbuf.at[k, pl.ds(dt * 8, 8)],
                out_ref.at[0, dt, 0],
                osems.at[k],
            ).wait()

    for k in range(_K):
        start_gather(k, k)

    @pl.loop(0, seq // _K)
    def _rounds(r):
        for k in range(_K):
            s = r * _K + k
            wait_gather(k, s)

            @pl.when(s >= _K)
            def _():
                wait_out(k)

            @pl.loop(0, _D // _LANES)
            def _tp(dc):
                dbase = dc * _LANES
                dvecs = [rots[j] + dbase for j in range(_LANES)]
                for c in range(bpw // _LANES):
                    ivec = iota + c * _LANES
                    for j in range(_LANES):
                        v = plsc.load_gather(gbuf.at[k], [ivec, dvecs[j]])
                        plsc.store_scatter(
                            tbuf.at[k], [dvecs[j], ivec], v * _SCALE
                        )

            @pl.when(s + _K < seq)
            def _():
                start_gather(k, s + _K)

            start_out(k, s)

    for k in range(_K):
        wait_out(k)


@jax.jit
def kernel(x, table):
    batch, seq = x.shape
    vocab = table.shape[0]
    bpw = batch // _NW

    xt = jnp.transpose(x).astype(jnp.int32) * 2
    tpad = jnp.pad(table, ((0, 0), (0, 128 - _D)))
    t2 = tpad.reshape(2 * vocab, _D)

    fn = pl.kernel(
        _body,
        out_type=jax.ShapeDtypeStruct((seq, _D // 8, _NW, 8, bpw), jnp.float32),
        mesh=plsc.VectorSubcoreMesh(core_axis_name="c", subcore_axis_name="s"),
        scratch_types=[
            pltpu.VMEM((seq, bpw), jnp.int32),
            pltpu.VMEM((_K, bpw, _D), jnp.float32),
            pltpu.VMEM((_K, _D, bpw), jnp.float32),
            pltpu.SemaphoreType.DMA((_K,)),
            pltpu.SemaphoreType.DMA((_K,)),
        ],
        compiler_params=pltpu.CompilerParams(
            use_tc_tiling_on_sc=False, needs_layout_passes=False
        ),
    )
    o5 = fn(xt, t2)
    return o5.transpose((2, 4, 0, 1, 3)).reshape(batch, seq, _D)


# speedup vs baseline: 3.1322x; 1.5900x over previous
import math

import jax
import jax.numpy as jnp
from jax import lax
from jax.experimental import pallas as pl
from jax.experimental.pallas import tpu as pltpu
from jax.experimental.pallas import tpu_sc as plsc

_D = 64
_LANES = 16
_NC = 2
_NS = 16
_NW = _NC * _NS
_K = 4
_SCALE = math.sqrt(_D)


def _body(xt_ref, table_ref, out_ref, idx_v, gbuf, tbuf, gsems, osems):
    wid = lax.axis_index("s") * _NC + lax.axis_index("c")
    seq = xt_ref.shape[0]
    bpw = xt_ref.shape[1] // _NW
    b0 = wid * bpw

    pltpu.sync_copy(xt_ref.at[:, pl.ds(b0, bpw)], idx_v)

    iota = lax.iota(jnp.int32, _LANES)
    rots = [jnp.bitwise_and(iota + j, _LANES - 1) for j in range(_LANES)]

    def start_gather(k, s):
        pltpu.async_copy(table_ref.at[idx_v.at[s]], gbuf.at[k], gsems.at[k])

    def wait_gather(k, s):
        pltpu.make_async_copy(
            table_ref.at[idx_v.at[s]], gbuf.at[k], gsems.at[k]
        ).wait()

    def start_out(k, s):
        for dt in range(_D // 8):
            pltpu.async_copy(
                tbuf.at[k, pl.ds(dt * 8, 8)],
                out_ref.at[s, dt, wid],
                osems.at[k],
            )

    def wait_out(k):
        for dt in range(_D // 8):
            pltpu.make_async_copy(
                tbuf.at[k, pl.ds(dt * 8, 8)],
                out_ref.at[0, dt, 0],
                osems.at[k],
            ).wait()

    for k in range(_K):
        start_gather(k, k)

    @pl.loop(0, seq // _K)
    def _rounds(r):
        for k in range(_K):
            s = r * _K + k
            wait_gather(k, s)

            @pl.when(s >= _K)
            def _():
                wait_out(k)

            @pl.loop(0, _D // _LANES)
            def _tp(dc):
                dbase = dc * _LANES
                dvecs = [rots[j] + dbase for j in range(_LANES)]
                for c in range(bpw // _LANES):
                    ivec = iota + c * _LANES
                    vals = [
                        plsc.load_gather(gbuf.at[k], [ivec, dvecs[j]])
                        for j in range(_LANES)
                    ]
                    for j in range(_LANES):
                        plsc.store_scatter(
                            tbuf.at[k], [dvecs[j], ivec], vals[j] * _SCALE
                        )

            @pl.when(s + _K < seq)
            def _():
                start_gather(k, s + _K)

            start_out(k, s)

    for k in range(_K):
        wait_out(k)


@jax.jit
def kernel(x, table):
    batch, seq = x.shape
    vocab = table.shape[0]
    bpw = batch // _NW

    xt = jnp.transpose(x).astype(jnp.int32) * 2
    tpad = jnp.pad(table, ((0, 0), (0, 128 - _D)))
    t2 = tpad.reshape(2 * vocab, _D)

    fn = pl.kernel(
        _body,
        out_type=jax.ShapeDtypeStruct((seq, _D // 8, _NW, 8, bpw), jnp.float32),
        mesh=plsc.VectorSubcoreMesh(core_axis_name="c", subcore_axis_name="s"),
        scratch_types=[
            pltpu.VMEM((seq, bpw), jnp.int32),
            pltpu.VMEM((_K, bpw, _D), jnp.float32),
            pltpu.VMEM((_K, _D, bpw), jnp.float32),
            pltpu.SemaphoreType.DMA((_K,)),
            pltpu.SemaphoreType.DMA((_K,)),
        ],
        compiler_params=pltpu.CompilerParams(
            use_tc_tiling_on_sc=False, needs_layout_passes=False
        ),
    )
    o5 = fn(xt, t2)
    return o5.transpose((2, 4, 0, 1, 3)).reshape(batch, seq, _D)
